# baseline (speedup 1.0000x reference)
import functools

import jax
import jax.numpy as jnp
from jax import lax
from jax.experimental import pallas as pl
from jax.experimental.pallas import tpu as pltpu
from jax.experimental.pallas import tpu_sc as plsc

N_USERS = 100000
N_ITEMS = 100000
DIM = 128
BATCH = 16384

LANES = 16
NUM_WORKERS = 32
B_PER_W = BATCH // NUM_WORKERS
CHUNK = 128
N_CHUNKS = B_PER_W // CHUNK


NBUF = 4


def _mf_kernel(user_table, item_table, user_list, pos_items, neg_items,
               user_out, pos_out, neg_out, partials,
               idx_all, bufs, acc_v,
               isem, g0, g1, g2, g3, w0, w1, w2, w3):
    nc = 2
    wid = lax.axis_index("s") * nc + lax.axis_index("c")
    base = wid * B_PER_W
    gsem = (g0, g1, g2, g3)
    wsem = (w0, w1, w2, w3)

    crow = wid * N_CHUNKS
    chunks = []
    for table, idx_hbm, out_hbm in (
        (user_table, user_list, user_out),
        (item_table, pos_items, pos_out),
        (item_table, neg_items, neg_out),
    ):
        for c in range(N_CHUNKS):
            chunks.append((table, idx_hbm, out_hbm, base + c * CHUNK))

    idescs = [
        pltpu.async_copy(idx_hbm.at[pl.ds(crow, N_CHUNKS)],
                         idx_all.at[pl.ds(t * N_CHUNKS, N_CHUNKS)], isem)
        for t, (_, idx_hbm, _, _) in enumerate(chunks[::N_CHUNKS])
    ]
    for d in idescs:
        d.wait()

    def gather(g, b):
        table = chunks[g][0]
        return pltpu.async_copy(table.at[idx_all.at[g]], bufs.at[b], gsem[b])

    gdescs = [None] * len(chunks)
    for g in range(NBUF):
        gdescs[g] = gather(g, g)

    accs = tuple(jnp.zeros((LANES,), jnp.float32) for _ in range(4))

    for g in range(len(chunks)):
        b = g % NBUF
        out_hbm, off = chunks[g][2], chunks[g][3]
        gdescs[g].wait()
        wdesc = pltpu.async_copy(bufs.at[b], out_hbm.at[pl.ds(off, CHUNK)],
                                 wsem[b])

        def body(r, xs, b=b):
            xs = list(xs)
            for rr in range(2):
                for cc in range(8):
                    v = bufs[b, 2 * r + rr, pl.ds(cc * LANES, LANES)]
                    xs[cc % 4] = xs[cc % 4] + v * v
            return tuple(xs)

        accs = lax.fori_loop(0, CHUNK // 2, body, accs)
        wdesc.wait()
        if g + NBUF < len(chunks):
            gdescs[g + NBUF] = gather(g + NBUF, b)

    acc_v[...] = (accs[0] + accs[1]) + (accs[2] + accs[3])
    pltpu.sync_copy(acc_v, partials.at[wid])


@jax.jit
def kernel(user_table, item_table, user_list, pos_items, neg_items):
    mesh = plsc.VectorSubcoreMesh(core_axis_name="c", subcore_axis_name="s")
    f = functools.partial(
        pl.kernel,
        mesh=mesh,
        out_type=[
            jax.ShapeDtypeStruct((BATCH, DIM), jnp.float32),
            jax.ShapeDtypeStruct((BATCH, DIM), jnp.float32),
            jax.ShapeDtypeStruct((BATCH, DIM), jnp.float32),
            jax.ShapeDtypeStruct((NUM_WORKERS, LANES), jnp.float32),
        ],
        scratch_types=[
            pltpu.VMEM((3 * N_CHUNKS, CHUNK), jnp.int32),
            pltpu.VMEM((NBUF, CHUNK, DIM), jnp.float32),
            pltpu.VMEM((LANES,), jnp.float32),
        ] + [pltpu.SemaphoreType.DMA] * 9,
    )(_mf_kernel)
    user_emb, posI_emb, negI_emb, partials = f(
        user_table, item_table,
        user_list.astype(jnp.int32).reshape(BATCH // CHUNK, CHUNK),
        pos_items.astype(jnp.int32).reshape(BATCH // CHUNK, CHUNK),
        neg_items.astype(jnp.int32).reshape(BATCH // CHUNK, CHUNK),
    )
    reg = jnp.sum(partials) / jnp.float32(BATCH)
    return (user_emb, posI_emb, negI_emb, reg)


# speedup vs baseline: 1.0324x; 1.0324x over previous
import functools

import jax
import jax.numpy as jnp
from jax import lax
from jax.experimental import pallas as pl
from jax.experimental.pallas import tpu as pltpu
from jax.experimental.pallas import tpu_sc as plsc

N_USERS = 100000
N_ITEMS = 100000
DIM = 128
BATCH = 16384

LANES = 16
NUM_WORKERS = 32
B_PER_W = BATCH // NUM_WORKERS
CHUNK = 128
N_CHUNKS = B_PER_W // CHUNK


NBUF = 3
GROUP = 2 * CHUNK
N_GROUPS = 3 * B_PER_W // GROUP


def _mf_kernel(user_table, item_table, user_list, pos_items, neg_items,
               user_out, pos_out, neg_out, partials,
               idx_all, bufs, acc_v,
               isem, g0, g1, g2, w0, w1, w2):
    nc = 2
    wid = lax.axis_index("s") * nc + lax.axis_index("c")
    base = wid * B_PER_W
    gsem = (g0, g1, g2)
    wsem = (w0, w1, w2)

    tables = (user_table, item_table, item_table)
    idxs = (user_list, pos_items, neg_items)
    outs = (user_out, pos_out, neg_out)

    crow = wid * N_CHUNKS

    idescs = [
        pltpu.async_copy(idx_hbm.at[pl.ds(crow, N_CHUNKS)],
                         idx_all.at[pl.ds(t * N_CHUNKS, N_CHUNKS)], isem)
        for t, idx_hbm in enumerate(idxs)
    ]
    for d in idescs:
        d.wait()

    def fire(grp):
        t, k = divmod(grp, 2)
        b = grp % NBUF
        return [
            pltpu.async_copy(
                tables[t].at[idx_all.at[t * N_CHUNKS + 2 * k + k2]],
                bufs.at[b, pl.ds(k2 * CHUNK, CHUNK)], gsem[b])
            for k2 in range(2)
        ]

    gdescs = [None] * N_GROUPS
    for grp in range(NBUF):
        gdescs[grp] = fire(grp)

    accs = tuple(jnp.zeros((LANES,), jnp.float32) for _ in range(4))

    for grp in range(N_GROUPS):
        t, k = divmod(grp, 2)
        b = grp % NBUF
        off = base + k * GROUP
        for d in gdescs[grp]:
            d.wait()
        wdesc = pltpu.async_copy(bufs.at[b], outs[t].at[pl.ds(off, GROUP)],
                                 wsem[b])

        def body(r, xs, b=b):
            xs = list(xs)
            for rr in range(2):
                for cc in range(8):
                    v = bufs[b, 2 * r + rr, pl.ds(cc * LANES, LANES)]
                    xs[cc % 4] = xs[cc % 4] + v * v
            return tuple(xs)

        accs = lax.fori_loop(0, GROUP // 2, body, accs)
        wdesc.wait()
        if grp + NBUF < N_GROUPS:
            gdescs[grp + NBUF] = fire(grp + NBUF)

    acc_v[...] = (accs[0] + accs[1]) + (accs[2] + accs[3])
    pltpu.sync_copy(acc_v, partials.at[wid])


@jax.jit
def kernel(user_table, item_table, user_list, pos_items, neg_items):
    mesh = plsc.VectorSubcoreMesh(core_axis_name="c", subcore_axis_name="s")
    f = functools.partial(
        pl.kernel,
        mesh=mesh,
        out_type=[
            jax.ShapeDtypeStruct((BATCH, DIM), jnp.float32),
            jax.ShapeDtypeStruct((BATCH, DIM), jnp.float32),
            jax.ShapeDtypeStruct((BATCH, DIM), jnp.float32),
            jax.ShapeDtypeStruct((NUM_WORKERS, LANES), jnp.float32),
        ],
        scratch_types=[
            pltpu.VMEM((3 * N_CHUNKS, CHUNK), jnp.int32),
            pltpu.VMEM((NBUF, GROUP, DIM), jnp.float32),
            pltpu.VMEM((LANES,), jnp.float32),
        ] + [pltpu.SemaphoreType.DMA] * 7,
    )(_mf_kernel)
    user_emb, posI_emb, negI_emb, partials = f(
        user_table, item_table,
        user_list.astype(jnp.int32).reshape(BATCH // CHUNK, CHUNK),
        pos_items.astype(jnp.int32).reshape(BATCH // CHUNK, CHUNK),
        neg_items.astype(jnp.int32).reshape(BATCH // CHUNK, CHUNK),
    )
    reg = jnp.sum(partials) / jnp.float32(BATCH)
    return (user_emb, posI_emb, negI_emb, reg)
